# baseline (speedup 1.0000x reference)
import jax
import jax.numpy as jnp
from jax import lax
from jax.experimental import pallas as pl
from jax.experimental.pallas import tpu as pltpu
from jax.experimental.pallas import tpu_sc as plsc

DIM = 64
BATCH = 16384
NC = 2
NS = 16
NW = NC * NS
B_PER_W = BATCH // NW


def _body(idx_hbm, w0_hbm, w1_hbm, out0_hbm, out1_hbm,
          idx_v, idx_sem, g0_sem, g1_sem):
    wid = lax.axis_index("s") * NC + lax.axis_index("c")
    base = wid * B_PER_W
    pltpu.async_copy(idx_hbm.at[pl.ds(base, B_PER_W)], idx_v, idx_sem).wait()

    def step(g, _):
        v = idx_v[pl.ds(g * 16, 16)]
        for j in range(16):
            s = v[j]
            i = g * 16 + j
            pltpu.async_copy(w0_hbm.at[pl.ds(s, 1), :],
                             out0_hbm.at[pl.ds(base + i, 1), :], g0_sem)
            pltpu.async_copy(w1_hbm.at[pl.ds(s, 1), :],
                             out1_hbm.at[pl.ds(base + i, 1), :], g1_sem)
        return ()

    lax.fori_loop(0, B_PER_W // 16, step, ())
    pltpu.make_async_copy(
        w0_hbm.at[pl.ds(0, B_PER_W), :],
        out0_hbm.at[pl.ds(base, B_PER_W), :], g0_sem).wait()
    pltpu.make_async_copy(
        w1_hbm.at[pl.ds(0, B_PER_W), :],
        out1_hbm.at[pl.ds(base, B_PER_W), :], g1_sem).wait()


@jax.jit
def _lookup(idx, W0, W1):
    mesh = plsc.VectorSubcoreMesh(core_axis_name="c", subcore_axis_name="s")
    run = pl.kernel(
        _body,
        mesh=mesh,
        out_type=(
            jax.ShapeDtypeStruct((BATCH, DIM), jnp.float32),
            jax.ShapeDtypeStruct((BATCH, DIM), jnp.float32),
        ),
        scratch_types=[
            pltpu.VMEM((B_PER_W,), jnp.int32),
            pltpu.SemaphoreType.DMA,
            pltpu.SemaphoreType.DMA,
            pltpu.SemaphoreType.DMA,
        ],
    )
    return run(idx, W0, W1)


def kernel(idx, W0, W1):
    e0, e1 = _lookup(idx.astype(jnp.int32), W0, W1)
    return (e0, e1)


# speedup vs baseline: 1.2896x; 1.2896x over previous
import jax
import jax.numpy as jnp
from jax import lax
from jax.experimental import pallas as pl
from jax.experimental.pallas import tpu as pltpu
from jax.experimental.pallas import tpu_sc as plsc

DIM = 64
BATCH = 16384
NC = 2
NS = 16
NW = NC * NS
B_PER_W = BATCH // NW


def _body(idx_hbm, wc_hbm, out_hbm, idx_v, rows_v, isem, gsem, wsem):
    wid = lax.axis_index("s") * NC + lax.axis_index("c")
    base = wid * B_PER_W
    pltpu.async_copy(idx_hbm.at[pl.ds(base, B_PER_W)], idx_v, isem).wait()
    pltpu.async_copy(wc_hbm.at[idx_v], rows_v, gsem).wait()
    pltpu.async_copy(rows_v, out_hbm.at[pl.ds(base, B_PER_W)], wsem).wait()


@jax.jit
def _lookup(idx, W0, W1):
    mesh = plsc.VectorSubcoreMesh(core_axis_name="c", subcore_axis_name="s")
    run = pl.kernel(
        _body,
        mesh=mesh,
        out_type=jax.ShapeDtypeStruct((BATCH, 2 * DIM), jnp.float32),
        scratch_types=[
            pltpu.VMEM((B_PER_W,), jnp.int32),
            pltpu.VMEM((B_PER_W, 2 * DIM), jnp.float32),
            pltpu.SemaphoreType.DMA,
            pltpu.SemaphoreType.DMA,
            pltpu.SemaphoreType.DMA,
        ],
    )
    wc = jnp.concatenate([W0, W1], axis=1)
    out = run(idx, wc)
    return out[:, :DIM], out[:, DIM:]


def kernel(idx, W0, W1):
    e0, e1 = _lookup(idx.astype(jnp.int32), W0, W1)
    return (e0, e1)
